# baseline (speedup 1.0000x reference)
import math

import jax
import jax.numpy as jnp
from jax import lax
from jax.experimental import pallas as pl
from jax.experimental.pallas import tpu as pltpu
from jax.experimental.pallas import tpu_sc as plsc

D_MODEL = 128
PATCH_LEN = 32
STRIDE = 16
KEEP = 96
DPOS = 32
_SIN_PI = math.sin(math.pi)
_COS_PI = math.cos(math.pi)

NC, NS, L = 2, 16, 16
B, S, V = 16, 4096, 12
NP = (S - PATCH_LEN) // STRIDE + 1
HALF_P = 128
NSAMP = HALF_P * STRIDE + STRIDE
NGRP = NSAMP // L
XWORDS = NSAMP * V
H1_OFF = (S - NSAMP) * V


def _sc_body(xf_hbm, w_hbm, b96_hbm, aux_hbm,
             comb_hbm, pos_hbm, len_hbm,
             xbuf, xm, wbuf, b96buf, auxbuf, olocal, posbuf, lenbuf):
    cid = lax.axis_index("c")
    sid = lax.axis_index("s")
    wid = sid * NC + cid
    bidx = wid // 2
    h = wid % 2

    iota = lax.iota(jnp.int32, L)

    pltpu.sync_copy(xf_hbm.at[bidx, pl.ds(h * H1_OFF, XWORDS)], xbuf)
    pltpu.sync_copy(w_hbm, wbuf)
    pltpu.sync_copy(b96_hbm, b96buf)
    pltpu.sync_copy(aux_hbm, auxbuf)

    idx_base = iota * V

    def mean_body(g, carry):
        base = g * (L * V)
        acc = plsc.load_gather(xbuf, [idx_base + base])
        for v in range(1, V):
            acc = acc + plsc.load_gather(xbuf, [idx_base + (base + v)])
        xm[pl.ds(g * L, L)] = acc * (1.0 / V)
        return carry

    lax.fori_loop(0, NGRP, mean_body, 0, unroll=2)

    pe0 = (auxbuf[0, pl.ds(0, L)] * _SIN_PI + auxbuf[1, pl.ds(0, L)] * _COS_PI
           + auxbuf[2, pl.ds(0, L)])
    pe1 = (auxbuf[0, pl.ds(L, L)] * _SIN_PI + auxbuf[1, pl.ds(L, L)] * _COS_PI
           + auxbuf[2, pl.ds(L, L)])
    for r in range(L):
        olocal[r, pl.ds(KEEP, L)] = pe0
        olocal[r, pl.ds(KEEP + L, L)] = pe1

    p_base = h * HALF_P

    for g in range(HALF_P // L):
        goff = g * (L * STRIDE) + h * STRIDE
        gvec = [plsc.load_gather(xm, [iota * STRIDE + (goff + l)])
                for l in range(PATCH_LEN)]

        def d_body(d, carry):
            acc = plsc.load_gather(b96buf, [jnp.full((L,), d, jnp.int32)])
            for l in range(PATCH_LEN):
                wsp = plsc.load_gather(
                    wbuf, [jnp.full((L,), d * PATCH_LEN + l, jnp.int32)])
                acc = acc + gvec[l] * wsp
            plsc.store_scatter(olocal,
                               [iota, jnp.full((L,), d, jnp.int32)], acc)
            return carry

        lax.fori_loop(0, KEEP, d_body, 0)

        p0 = p_base + g * L
        if g < HALF_P // L - 1:
            pltpu.sync_copy(olocal, comb_hbm.at[bidx, pl.ds(p0, L), :])
        else:
            @pl.when(h == 0)
            def _():
                pltpu.sync_copy(olocal, comb_hbm.at[bidx, pl.ds(p0, L), :])

            @pl.when(h == 1)
            def _():
                pltpu.sync_copy(olocal.at[pl.ds(0, L - 1), :],
                                comb_hbm.at[bidx, pl.ds(p0, L - 1), :])

        posbuf[pl.ds(g * L, L)] = (iota + (p0 + 1)).astype(jnp.float32) * 16.0
        lenbuf[pl.ds(g * L, L)] = jnp.full((L,), float(PATCH_LEN), jnp.float32)

    @pl.when(h == 0)
    def _():
        pltpu.sync_copy(posbuf, pos_hbm.at[bidx, pl.ds(0, HALF_P)])
        pltpu.sync_copy(lenbuf, len_hbm.at[bidx, pl.ds(0, HALF_P)])

    @pl.when(h == 1)
    def _():
        pltpu.sync_copy(posbuf.at[pl.ds(0, HALF_P - 1)],
                        pos_hbm.at[bidx, pl.ds(HALF_P, HALF_P - 1)])
        pltpu.sync_copy(lenbuf.at[pl.ds(0, HALF_P - 1)],
                        len_hbm.at[bidx, pl.ds(HALF_P, HALF_P - 1)])


def _make_sc_kernel():
    mesh = plsc.VectorSubcoreMesh(core_axis_name="c", subcore_axis_name="s",
                                  num_cores=NC, num_subcores=NS)
    return pl.kernel(
        _sc_body,
        out_type=(
            jax.ShapeDtypeStruct((B, NP, D_MODEL), jnp.float32),
            jax.ShapeDtypeStruct((B, NP), jnp.float32),
            jax.ShapeDtypeStruct((B, NP), jnp.float32),
        ),
        mesh=mesh,
        compiler_params=pltpu.CompilerParams(use_tc_tiling_on_sc=False,
                                             needs_layout_passes=False),
        scratch_types=[
            pltpu.VMEM((XWORDS,), jnp.float32),
            pltpu.VMEM((NGRP * L + L,), jnp.float32),
            pltpu.VMEM((KEEP * PATCH_LEN,), jnp.float32),
            pltpu.VMEM((KEEP,), jnp.float32),
            pltpu.VMEM((3, DPOS), jnp.float32),
            pltpu.VMEM((L, D_MODEL), jnp.float32),
            pltpu.VMEM((HALF_P,), jnp.float32),
            pltpu.VMEM((HALF_P,), jnp.float32),
        ],
    )


def kernel(x, W, b, W2, b2):
    xf = x.reshape(B, S * V)
    wflat = W[:KEEP].reshape(KEEP * PATCH_LEN)
    b96 = b[:KEEP]
    aux = jnp.stack([W2[:, 0], W2[:, 1], b2])
    return _make_sc_kernel()(xf, wflat, b96, aux)


# speedup vs baseline: 1.0509x; 1.0509x over previous
import math

import jax
import jax.numpy as jnp
from jax import lax
from jax.experimental import pallas as pl
from jax.experimental.pallas import tpu as pltpu
from jax.experimental.pallas import tpu_sc as plsc

D_MODEL = 128
PATCH_LEN = 32
STRIDE = 16
KEEP = 96
DPOS = 32
_SIN_PI = math.sin(math.pi)
_COS_PI = math.cos(math.pi)

NC, NS, L = 2, 16, 16
B, S, V = 16, 4096, 12
NP = (S - PATCH_LEN) // STRIDE + 1
HALF_P = 128
NSAMP = HALF_P * STRIDE + STRIDE
NGRP = NSAMP // L
H1_ROW = S - NSAMP
CHUNK = 688


def _sc_body(x_hbm, w_hbm, b96_hbm, aux_hbm, comb_hbm,
             xbuf, xm, wbuf, b96buf, auxbuf, olocal):
    cid = lax.axis_index("c")
    sid = lax.axis_index("s")
    wid = sid * NC + cid
    bidx = wid // 2
    h = wid % 2

    iota = lax.iota(jnp.int32, L)

    row0 = bidx * S + h * H1_ROW
    pltpu.sync_copy(w_hbm, wbuf)
    pltpu.sync_copy(b96_hbm, b96buf)
    pltpu.sync_copy(aux_hbm, auxbuf)

    for c in range(NSAMP // CHUNK):
        pltpu.sync_copy(x_hbm.at[pl.ds(row0 + c * CHUNK, CHUNK), :], xbuf)

        def mean_body(g, carry):
            rows = iota + g * L
            acc = plsc.load_gather(xbuf, [rows, jnp.zeros((L,), jnp.int32)])
            for v in range(1, V):
                acc = acc + plsc.load_gather(
                    xbuf, [rows, jnp.full((L,), v, jnp.int32)])
            xm[pl.ds(c * CHUNK + g * L, L)] = acc * (1.0 / V)
            return carry

        lax.fori_loop(0, CHUNK // L, mean_body, 0, unroll=2)

    pe0 = (auxbuf[pl.ds(0, L)] * _SIN_PI + auxbuf[pl.ds(DPOS, L)] * _COS_PI
           + auxbuf[pl.ds(2 * DPOS, L)])
    pe1 = (auxbuf[pl.ds(L, L)] * _SIN_PI + auxbuf[pl.ds(DPOS + L, L)] * _COS_PI
           + auxbuf[pl.ds(2 * DPOS + L, L)])
    for r in range(L):
        olocal[r, pl.ds(KEEP, L)] = pe0
        olocal[r, pl.ds(KEEP + L, L)] = pe1

    p_base = h * HALF_P

    for g in range(HALF_P // L):
        goff = g * (L * STRIDE) + h * STRIDE
        base_idx = iota * STRIDE + goff
        gvec = [plsc.load_gather(xm, [base_idx + l]) for l in range(PATCH_LEN)]

        def d_body(d, carry):
            acc = plsc.load_gather(b96buf, [jnp.full((L,), d, jnp.int32)])
            wi = jnp.full((L,), d * PATCH_LEN, jnp.int32)
            for l in range(PATCH_LEN):
                wsp = plsc.load_gather(wbuf, [wi + l])
                acc = acc + gvec[l] * wsp
            plsc.store_scatter(olocal,
                               [iota, jnp.full((L,), d, jnp.int32)], acc)
            return carry

        lax.fori_loop(0, KEEP, d_body, 0, unroll=2)

        p0 = p_base + g * L
        pltpu.sync_copy(olocal, comb_hbm.at[bidx, pl.ds(p0, L), :])


def _make_sc_kernel():
    mesh = plsc.VectorSubcoreMesh(core_axis_name="c", subcore_axis_name="s",
                                  num_cores=NC, num_subcores=NS)
    return pl.kernel(
        _sc_body,
        out_type=jax.ShapeDtypeStruct((B, NP + 1, D_MODEL), jnp.float32),
        mesh=mesh,
        compiler_params=pltpu.CompilerParams(use_tc_tiling_on_sc=True,
                                             needs_layout_passes=False),
        scratch_types=[
            pltpu.VMEM((CHUNK, V), jnp.float32),
            pltpu.VMEM((NGRP * L + L,), jnp.float32),
            pltpu.VMEM((KEEP * PATCH_LEN,), jnp.float32),
            pltpu.VMEM((KEEP,), jnp.float32),
            pltpu.VMEM((3 * DPOS,), jnp.float32),
            pltpu.VMEM((L, D_MODEL), jnp.float32),
        ],
    )


def kernel(x, W, b, W2, b2):
    x2d = x.reshape(B * S, V)
    wflat = W[:KEEP].reshape(KEEP * PATCH_LEN)
    b96 = b[:KEEP]
    aux = jnp.concatenate([W2[:, 0], W2[:, 1], b2])
    comb = _make_sc_kernel()(x2d, wflat, b96, aux)[:, :NP, :]
    centers = jnp.arange(NP, dtype=jnp.float32) * STRIDE + PATCH_LEN // 2
    positions = jnp.broadcast_to(centers, (B, NP))
    patch_lens = jnp.full((B, NP), float(PATCH_LEN), jnp.float32)
    return comb, positions, patch_lens


# speedup vs baseline: 1.4748x; 1.4033x over previous
import math

import jax
import jax.numpy as jnp
from jax import lax
from jax.experimental import pallas as pl
from jax.experimental.pallas import tpu as pltpu
from jax.experimental.pallas import tpu_sc as plsc

D_MODEL = 128
PATCH_LEN = 32
STRIDE = 16
KEEP = 96
DPOS = 32
_SIN_PI = math.sin(math.pi)
_COS_PI = math.cos(math.pi)

NC, NS, L = 2, 16, 16
B, S, V = 16, 4096, 12
NP = (S - PATCH_LEN) // STRIDE + 1
HALF_P = 128
NSAMP = HALF_P * STRIDE + STRIDE
NGRP = NSAMP // L
H1_ROW = S - NSAMP
CHUNK = 688
NDC = KEEP // L


def _sc_body(x_hbm, w_hbm, b96_hbm, aux_hbm, comb_hbm,
             xbuf, xm, wbuf, b96buf, auxbuf, olocal):
    cid = lax.axis_index("c")
    sid = lax.axis_index("s")
    wid = sid * NC + cid
    bidx = wid // 2
    h = wid % 2

    iota = lax.iota(jnp.int32, L)

    row0 = h * H1_ROW
    pltpu.sync_copy(w_hbm, wbuf)
    pltpu.sync_copy(b96_hbm, b96buf)
    pltpu.sync_copy(aux_hbm, auxbuf)

    for c in range(NSAMP // CHUNK):
        pltpu.sync_copy(x_hbm.at[bidx, pl.ds(row0 + c * CHUNK, CHUNK), :],
                        xbuf)

        def mean_body(g, carry):
            rows = iota + g * L
            acc = plsc.load_gather(xbuf, [rows, jnp.zeros((L,), jnp.int32)])
            for v in range(1, V):
                acc = acc + plsc.load_gather(
                    xbuf, [rows, jnp.full((L,), v, jnp.int32)])
            xm[pl.ds(c * CHUNK + g * L, L)] = acc * (1.0 / V)
            return carry

        lax.fori_loop(0, CHUNK // L, mean_body, 0, unroll=2)

    pe0 = (auxbuf[pl.ds(0, L)] * _SIN_PI + auxbuf[pl.ds(DPOS, L)] * _COS_PI
           + auxbuf[pl.ds(2 * DPOS, L)])
    pe1 = (auxbuf[pl.ds(L, L)] * _SIN_PI + auxbuf[pl.ds(DPOS + L, L)] * _COS_PI
           + auxbuf[pl.ds(2 * DPOS + L, L)])
    bvec = [b96buf[pl.ds(j * L, L)] for j in range(NDC)]

    p_base = h * HALF_P

    for g in range(4):
        loff_a = g * (L * STRIDE) + h * STRIDE
        loff_b = loff_a + 4 * (L * STRIDE)

        def patch_body(pi, carry):
            sa = jnp.full((L,), loff_a, jnp.int32) + pi * STRIDE
            sb = jnp.full((L,), loff_b, jnp.int32) + pi * STRIDE
            acc_a = list(bvec)
            acc_b = list(bvec)
            for l in range(PATCH_LEN):
                xsa = plsc.load_gather(xm, [sa + l])
                xsb = plsc.load_gather(xm, [sb + l])
                for j in range(NDC):
                    wv = wbuf[pl.ds(l * KEEP + j * L, L)]
                    acc_a[j] = acc_a[j] + xsa * wv
                    acc_b[j] = acc_b[j] + xsb * wv
            for j in range(NDC):
                olocal[pi, pl.ds(j * L, L)] = acc_a[j]
                olocal[pi + L, pl.ds(j * L, L)] = acc_b[j]
            olocal[pi, pl.ds(KEEP, L)] = pe0
            olocal[pi, pl.ds(KEEP + L, L)] = pe1
            olocal[pi + L, pl.ds(KEEP, L)] = pe0
            olocal[pi + L, pl.ds(KEEP + L, L)] = pe1
            return carry

        lax.fori_loop(0, L, patch_body, 0)

        pltpu.sync_copy(olocal.at[pl.ds(0, L), :],
                        comb_hbm.at[bidx, pl.ds(p_base + g * L, L), :])
        pltpu.sync_copy(olocal.at[pl.ds(L, L), :],
                        comb_hbm.at[bidx, pl.ds(p_base + (g + 4) * L, L), :])


def _make_sc_kernel():
    mesh = plsc.VectorSubcoreMesh(core_axis_name="c", subcore_axis_name="s",
                                  num_cores=NC, num_subcores=NS)
    return pl.kernel(
        _sc_body,
        out_type=jax.ShapeDtypeStruct((B, NP + 1, D_MODEL), jnp.float32),
        mesh=mesh,
        compiler_params=pltpu.CompilerParams(use_tc_tiling_on_sc=True,
                                             needs_layout_passes=False),
        scratch_types=[
            pltpu.VMEM((CHUNK, V), jnp.float32),
            pltpu.VMEM((NGRP * L + L,), jnp.float32),
            pltpu.VMEM((PATCH_LEN * KEEP,), jnp.float32),
            pltpu.VMEM((KEEP,), jnp.float32),
            pltpu.VMEM((3 * DPOS,), jnp.float32),
            pltpu.VMEM((2 * L, D_MODEL), jnp.float32),
        ],
    )


def kernel(x, W, b, W2, b2):
    wT = W[:KEEP].T.reshape(PATCH_LEN * KEEP)
    b96 = b[:KEEP]
    aux = jnp.concatenate([W2[:, 0], W2[:, 1], b2])
    comb = _make_sc_kernel()(x, wT, b96, aux)[:, :NP, :]
    centers = jnp.arange(NP, dtype=jnp.float32) * STRIDE + PATCH_LEN // 2
    positions = jnp.broadcast_to(centers, (B, NP))
    patch_lens = jnp.full((B, NP), float(PATCH_LEN), jnp.float32)
    return comb, positions, patch_lens
